# baseline (speedup 1.0000x reference)
import jax
import jax.numpy as jnp
from jax.experimental import pallas as pl
from jax.experimental.pallas import tpu as pltpu

BATCH = 128
D_MODEL = 1024
VOCAB = 100000
TILE_V = 2048


def _linear_tile(x_ref, wa_ref, wb_ref, b_ref, o_ref):
    xb = x_ref[...]
    acc_a = jax.lax.dot_general(
        xb, wa_ref[...].astype(jnp.bfloat16),
        dimension_numbers=(((1,), (1,)), ((), ())),
        preferred_element_type=jnp.float32,
    )
    acc_b = jax.lax.dot_general(
        xb, wb_ref[...].astype(jnp.bfloat16),
        dimension_numbers=(((1,), (1,)), ((), ())),
        preferred_element_type=jnp.float32,
    )
    bias = b_ref[...]
    o_ref[:, :TILE_V] = acc_a + bias[:, :TILE_V]
    o_ref[:, TILE_V:] = acc_b + bias[:, TILE_V:]


def kernel(x, W, b):
    xb = x.astype(jnp.bfloat16)
    b2 = b.reshape(1, VOCAB)
    grid = (pl.cdiv(VOCAB, 2 * TILE_V),)
    out = pl.pallas_call(
        _linear_tile,
        grid=grid,
        in_specs=[
            pl.BlockSpec((BATCH, D_MODEL), lambda i: (0, 0)),
            pl.BlockSpec((TILE_V, D_MODEL), lambda i: (2 * i, 0)),
            pl.BlockSpec((TILE_V, D_MODEL), lambda i: (jnp.minimum(2 * i + 1, VOCAB // TILE_V), 0)),
            pl.BlockSpec((1, 2 * TILE_V), lambda i: (0, i)),
        ],
        out_specs=pl.BlockSpec((BATCH, 2 * TILE_V), lambda i: (0, i)),
        out_shape=jax.ShapeDtypeStruct((BATCH, VOCAB), jnp.float32),
        compiler_params=pltpu.CompilerParams(
            dimension_semantics=("arbitrary",),
        ),
    )(xb, W, W, b2)
    return out
